# Initial kernel scaffold; baseline (speedup 1.0000x reference)
import jax
import jax.numpy as jnp
from jax.experimental import pallas as pl


def kernel(x, edge_index, W1_rel, b1_rel, W1_root, W2_rel, b2_rel, W2_root, W3, b3, W4, b4, W5, b5):
    raise NotImplementedError("write your pallas kernel here")


# speedup vs baseline: 2.7866x; 2.7866x over previous
import functools

import jax
import jax.numpy as jnp
from jax import lax
from jax.experimental import pallas as pl
from jax.experimental.pallas import tpu as pltpu
from jax.experimental.pallas import tpu_sc as plsc

_N = 10000
_C = 128
_E = 320000

_NW = 32
_CHUNK = 128
_CHUNKS = 80
_EPW = _CHUNK * _CHUNKS
_EPAD = _NW * _EPW
_RPT = 632
_NPAD = 16 * _RPT

_BLK = 1000


def _make_sc_agg():
    mesh = plsc.VectorSubcoreMesh(core_axis_name="c", subcore_axis_name="s")

    @functools.partial(
        pl.kernel,
        mesh=mesh,
        out_type=jax.ShapeDtypeStruct((2, _NPAD, _C), jnp.float32),
        scratch_types=[
            pltpu.VMEM((_CHUNKS, _CHUNK), jnp.int32),
            pltpu.VMEM((_CHUNKS, _CHUNK), jnp.int32),
            pltpu.VMEM((_CHUNK, _C), jnp.float32),
            pltpu.VMEM_SHARED((_NPAD, _C), jnp.float32),
            pltpu.SemaphoreType.DMA,
        ],
    )
    def sc_agg(x_hbm, src_hbm, dst_hbm, zeros_hbm, out_hbm,
               sidx, didx, rows, acc, sem):
        cid = lax.axis_index("c")
        sid = lax.axis_index("s")
        wid = cid * 16 + sid
        pltpu.sync_copy(src_hbm.at[wid], sidx)
        pltpu.sync_copy(dst_hbm.at[wid], didx)
        pltpu.sync_copy(zeros_hbm, acc.at[pl.ds(sid * _RPT, _RPT)])
        plsc.subcore_barrier()

        def body(j, carry):
            pltpu.async_copy(x_hbm.at[sidx.at[j]], rows, sem).wait()
            pltpu.sync_copy(rows, acc.at[didx.at[j]], add=True)
            return carry

        lax.fori_loop(0, _CHUNKS, body, 0)
        plsc.subcore_barrier()
        pltpu.sync_copy(acc.at[pl.ds(sid * _RPT, _RPT)],
                        out_hbm.at[cid, pl.ds(sid * _RPT, _RPT)])

    return sc_agg


_sc_agg = _make_sc_agg()


def _tc_layer1_body(p0, p1, xb, wrel, wroot, b, o):
    agg = p0[...] + p1[...]
    o[...] = jnp.maximum(
        jnp.dot(agg, wrel[...], preferred_element_type=jnp.float32)
        + jnp.dot(xb[...], wroot[...], preferred_element_type=jnp.float32)
        + b[...], 0.0)


def _tc_layer2_body(q0, q1, x1b, wrel, wroot, b2, w3a, w3b, b3, w4, b4,
                    w5, b5, o):
    agg = q0[...] + q1[...]
    x2 = jnp.maximum(
        jnp.dot(agg, wrel[...], preferred_element_type=jnp.float32)
        + jnp.dot(x1b[...], wroot[...], preferred_element_type=jnp.float32)
        + b2[...], 0.0)
    h = jnp.maximum(
        jnp.dot(x1b[...], w3a[...], preferred_element_type=jnp.float32)
        + jnp.dot(x2, w3b[...], preferred_element_type=jnp.float32)
        + b3[...], 0.0)
    h = jnp.maximum(
        jnp.dot(h, w4[...], preferred_element_type=jnp.float32)
        + b4[...], 0.0)
    o[...] = jnp.dot(h, w5[...], preferred_element_type=jnp.float32) + b5[...]


def _row_spec(cols):
    return pl.BlockSpec((_BLK, cols), lambda i: (i, 0))


def _full_spec(r, c):
    return pl.BlockSpec((r, c), lambda i: (0, 0))


def _tc_layer1(p0, p1, x, wrel_t, wroot_t, b):
    return pl.pallas_call(
        _tc_layer1_body,
        grid=(_N // _BLK,),
        in_specs=[
            _row_spec(_C), _row_spec(_C), _row_spec(_C),
            _full_spec(_C, _C), _full_spec(_C, _C), _full_spec(1, _C),
        ],
        out_specs=_row_spec(_C),
        out_shape=jax.ShapeDtypeStruct((_N, _C), jnp.float32),
    )(p0, p1, x, wrel_t, wroot_t, b)


def _tc_layer2(q0, q1, x1, wrel_t, wroot_t, b2, w3a, w3b, b3, w4, b4, w5, b5):
    return pl.pallas_call(
        _tc_layer2_body,
        grid=(_N // _BLK,),
        in_specs=[
            _row_spec(_C), _row_spec(_C), _row_spec(_C),
            _full_spec(_C, _C), _full_spec(_C, _C), _full_spec(1, _C),
            _full_spec(_C, 64), _full_spec(_C, 64), _full_spec(1, 64),
            _full_spec(64, 32), _full_spec(1, 32),
            _full_spec(32, 16), _full_spec(1, 16),
        ],
        out_specs=_row_spec(16),
        out_shape=jax.ShapeDtypeStruct((_N, 16), jnp.float32),
    )(q0, q1, x1, wrel_t, wroot_t, b2, w3a, w3b, b3, w4, b4, w5, b5)


def kernel(x, edge_index, W1_rel, b1_rel, W1_root, W2_rel, b2_rel, W2_root,
           W3, b3, W4, b4, W5, b5):
    src = edge_index[0]
    dst = edge_index[1]
    pad = _EPAD - _E
    src_p = jnp.concatenate([src, jnp.zeros((pad,), jnp.int32)])
    dst_p = jnp.concatenate([dst, jnp.full((pad,), _N, jnp.int32)])
    src_r = src_p.reshape(_NW, _CHUNKS, _CHUNK)
    dst_r = dst_p.reshape(_NW, _CHUNKS, _CHUNK)
    zeros = jnp.zeros((_RPT, _C), jnp.float32)

    parts1 = _sc_agg(x, src_r, dst_r, zeros)
    x1 = _tc_layer1(parts1[0, :_N], parts1[1, :_N], x,
                    W1_rel.T, W1_root.T, b1_rel[None, :])

    parts2 = _sc_agg(x1, src_r, dst_r, zeros)
    out = _tc_layer2(parts2[0, :_N], parts2[1, :_N], x1,
                     W2_rel.T, W2_root.T, b2_rel[None, :],
                     W3[:, :_C].T, W3[:, _C:].T, b3[None, :],
                     W4.T, b4[None, :], W5.T, b5[None, :])
    return out


# speedup vs baseline: 3.0833x; 1.1065x over previous
import functools

import jax
import jax.numpy as jnp
from jax import lax
from jax.experimental import pallas as pl
from jax.experimental.pallas import tpu as pltpu
from jax.experimental.pallas import tpu_sc as plsc

_N = 10000
_C = 128
_E = 320000

_NW = 32
_CHUNK = 128
_PHASE = 40
_NPHASE = 2
_CHUNKS = _PHASE * _NPHASE
_EPW = _CHUNK * _CHUNKS
_EPAD = _NW * _EPW
_RPT = 632
_NPAD = 16 * _RPT

_BLK = 1000


def _make_sc_agg():
    mesh = plsc.VectorSubcoreMesh(core_axis_name="c", subcore_axis_name="s")

    @functools.partial(
        pl.kernel,
        mesh=mesh,
        out_type=jax.ShapeDtypeStruct((2, _NPAD, _C), jnp.float32),
        scratch_types=[
            pltpu.VMEM((_PHASE, _CHUNK), jnp.int32),
            pltpu.VMEM((_PHASE, _CHUNK), jnp.int32),
            pltpu.VMEM((_CHUNK, _C), jnp.float32),
            pltpu.VMEM((_CHUNK, _C), jnp.float32),
            pltpu.VMEM_SHARED((_NPAD, _C), jnp.float32),
            pltpu.SemaphoreType.DMA,
            pltpu.SemaphoreType.DMA,
        ],
    )
    def sc_agg(x_hbm, src_hbm, dst_hbm, zeros_hbm, out_hbm,
               sidx, didx, rows0, rows1, acc, sem0, sem1):
        cid = lax.axis_index("c")
        sid = lax.axis_index("s")
        wid = cid * 16 + sid
        pltpu.sync_copy(zeros_hbm, acc.at[pl.ds(sid * _RPT, _RPT)])
        plsc.subcore_barrier()

        for p in range(_NPHASE):
            pltpu.sync_copy(src_hbm.at[wid, pl.ds(p * _PHASE, _PHASE)], sidx)
            pltpu.sync_copy(dst_hbm.at[wid, pl.ds(p * _PHASE, _PHASE)], didx)
            pltpu.async_copy(x_hbm.at[sidx.at[0]], rows0, sem0)
            pltpu.async_copy(x_hbm.at[sidx.at[1]], rows1, sem1)

            def body(i, carry):
                j = 2 * i
                pltpu.make_async_copy(x_hbm.at[sidx.at[j]], rows0, sem0).wait()
                pltpu.sync_copy(rows0, acc.at[didx.at[j]], add=True)

                @pl.when(j + 2 < _PHASE)
                def _():
                    pltpu.async_copy(x_hbm.at[sidx.at[j + 2]], rows0, sem0)

                pltpu.make_async_copy(
                    x_hbm.at[sidx.at[j + 1]], rows1, sem1).wait()
                pltpu.sync_copy(rows1, acc.at[didx.at[j + 1]], add=True)

                @pl.when(j + 3 < _PHASE)
                def _():
                    pltpu.async_copy(x_hbm.at[sidx.at[j + 3]], rows1, sem1)
                return carry

            lax.fori_loop(0, _PHASE // 2, body, 0)
        plsc.subcore_barrier()
        pltpu.sync_copy(acc.at[pl.ds(sid * _RPT, _RPT)],
                        out_hbm.at[cid, pl.ds(sid * _RPT, _RPT)])

    return sc_agg


_sc_agg = _make_sc_agg()


def _tc_layer1_body(p0, p1, xb, wrel, wroot, b, o):
    agg = p0[...] + p1[...]
    o[...] = jnp.maximum(
        jnp.dot(agg, wrel[...], preferred_element_type=jnp.float32)
        + jnp.dot(xb[...], wroot[...], preferred_element_type=jnp.float32)
        + b[...], 0.0)


def _tc_layer2_body(q0, q1, x1b, wrel, wroot, b2, w3a, w3b, b3, w4, b4,
                    w5, b5, o):
    agg = q0[...] + q1[...]
    x2 = jnp.maximum(
        jnp.dot(agg, wrel[...], preferred_element_type=jnp.float32)
        + jnp.dot(x1b[...], wroot[...], preferred_element_type=jnp.float32)
        + b2[...], 0.0)
    h = jnp.maximum(
        jnp.dot(x1b[...], w3a[...], preferred_element_type=jnp.float32)
        + jnp.dot(x2, w3b[...], preferred_element_type=jnp.float32)
        + b3[...], 0.0)
    h = jnp.maximum(
        jnp.dot(h, w4[...], preferred_element_type=jnp.float32)
        + b4[...], 0.0)
    o[...] = jnp.dot(h, w5[...], preferred_element_type=jnp.float32) + b5[...]


def _row_spec(cols):
    return pl.BlockSpec((_BLK, cols), lambda i: (i, 0))


def _full_spec(r, c):
    return pl.BlockSpec((r, c), lambda i: (0, 0))


def _tc_layer1(p0, p1, x, wrel_t, wroot_t, b):
    return pl.pallas_call(
        _tc_layer1_body,
        grid=(_N // _BLK,),
        in_specs=[
            _row_spec(_C), _row_spec(_C), _row_spec(_C),
            _full_spec(_C, _C), _full_spec(_C, _C), _full_spec(1, _C),
        ],
        out_specs=_row_spec(_C),
        out_shape=jax.ShapeDtypeStruct((_N, _C), jnp.float32),
    )(p0, p1, x, wrel_t, wroot_t, b)


def _tc_layer2(q0, q1, x1, wrel_t, wroot_t, b2, w3a, w3b, b3, w4, b4, w5, b5):
    return pl.pallas_call(
        _tc_layer2_body,
        grid=(_N // _BLK,),
        in_specs=[
            _row_spec(_C), _row_spec(_C), _row_spec(_C),
            _full_spec(_C, _C), _full_spec(_C, _C), _full_spec(1, _C),
            _full_spec(_C, 64), _full_spec(_C, 64), _full_spec(1, 64),
            _full_spec(64, 32), _full_spec(1, 32),
            _full_spec(32, 16), _full_spec(1, 16),
        ],
        out_specs=_row_spec(16),
        out_shape=jax.ShapeDtypeStruct((_N, 16), jnp.float32),
    )(q0, q1, x1, wrel_t, wroot_t, b2, w3a, w3b, b3, w4, b4, w5, b5)


def kernel(x, edge_index, W1_rel, b1_rel, W1_root, W2_rel, b2_rel, W2_root,
           W3, b3, W4, b4, W5, b5):
    src = edge_index[0]
    dst = edge_index[1]
    pad = _EPAD - _E
    src_p = jnp.concatenate([src, jnp.zeros((pad,), jnp.int32)])
    dst_p = jnp.concatenate([dst, jnp.full((pad,), _N, jnp.int32)])
    src_r = src_p.reshape(_NW, _CHUNKS, _CHUNK)
    dst_r = dst_p.reshape(_NW, _CHUNKS, _CHUNK)
    zeros = jnp.zeros((_RPT, _C), jnp.float32)

    parts1 = _sc_agg(x, src_r, dst_r, zeros)
    x1 = _tc_layer1(parts1[0, :_N], parts1[1, :_N], x,
                    W1_rel.T, W1_root.T, b1_rel[None, :])

    parts2 = _sc_agg(x1, src_r, dst_r, zeros)
    out = _tc_layer2(parts2[0, :_N], parts2[1, :_N], x1,
                     W2_rel.T, W2_root.T, b2_rel[None, :],
                     W3[:, :_C].T, W3[:, _C:].T, b3[None, :],
                     W4.T, b4[None, :], W5.T, b5[None, :])
    return out


# speedup vs baseline: 11.5472x; 3.7451x over previous
import functools

import jax
import jax.numpy as jnp
from jax import lax
from jax.experimental import pallas as pl
from jax.experimental.pallas import tpu as pltpu
from jax.experimental.pallas import tpu_sc as plsc

_N = 10000
_C = 128
_E = 320000

_NW = 32
_CHUNK = 128
_PHASE = 40
_NPHASE = 2
_CHUNKS = _PHASE * _NPHASE
_EPW = _CHUNK * _CHUNKS
_EPAD = _NW * _EPW
_RPT = 632
_NPAD = 16 * _RPT

_BLK = 1000


def _make_sc_agg():
    mesh = plsc.VectorSubcoreMesh(core_axis_name="c", subcore_axis_name="s")

    @functools.partial(
        pl.kernel,
        mesh=mesh,
        out_type=jax.ShapeDtypeStruct((2, _NPAD, _C), jnp.float32),
        scratch_types=[
            pltpu.VMEM((_PHASE, _CHUNK), jnp.int32),
            pltpu.VMEM((_PHASE, _CHUNK), jnp.int32),
            pltpu.VMEM((_CHUNK, _C), jnp.float32),
            pltpu.VMEM((_CHUNK, _C), jnp.float32),
            pltpu.VMEM_SHARED((_NPAD, _C), jnp.float32),
            pltpu.SemaphoreType.DMA,
            pltpu.SemaphoreType.DMA,
        ],
    )
    def sc_agg(x_hbm, src_hbm, dst_hbm, zeros_hbm, out_hbm,
               sidx, didx, rows0, rows1, acc, sem0, sem1):
        cid = lax.axis_index("c")
        sid = lax.axis_index("s")
        wid = cid * 16 + sid
        pltpu.sync_copy(zeros_hbm, acc.at[pl.ds(sid * _RPT, _RPT)])
        plsc.subcore_barrier()

        for p in range(_NPHASE):
            pltpu.sync_copy(src_hbm.at[wid, pl.ds(p * _PHASE, _PHASE)], sidx)
            pltpu.sync_copy(dst_hbm.at[wid, pl.ds(p * _PHASE, _PHASE)], didx)
            pltpu.async_copy(x_hbm.at[sidx.at[0]], rows0, sem0)
            pltpu.async_copy(x_hbm.at[sidx.at[1]], rows1, sem1)

            def body(i, carry):
                j = 2 * i
                pltpu.make_async_copy(x_hbm.at[sidx.at[j]], rows0, sem0).wait()
                pltpu.sync_copy(rows0, acc.at[didx.at[j]], add=True)

                @pl.when(j + 2 < _PHASE)
                def _():
                    pltpu.async_copy(x_hbm.at[sidx.at[j + 2]], rows0, sem0)

                pltpu.make_async_copy(
                    x_hbm.at[sidx.at[j + 1]], rows1, sem1).wait()
                pltpu.sync_copy(rows1, acc.at[didx.at[j + 1]], add=True)

                @pl.when(j + 3 < _PHASE)
                def _():
                    pltpu.async_copy(x_hbm.at[sidx.at[j + 3]], rows1, sem1)
                return carry

            lax.fori_loop(0, _PHASE // 2, body, 0)
        plsc.subcore_barrier()
        pltpu.sync_copy(acc.at[pl.ds(sid * _RPT, _RPT)],
                        out_hbm.at[cid, pl.ds(sid * _RPT, _RPT)])

    return sc_agg


_sc_agg = _make_sc_agg()


def _tc_layer1_body(p0, p1, xb, wrel, wroot, b, o):
    agg = p0[...] + p1[...]
    o[...] = jnp.maximum(
        jnp.dot(agg, wrel[...], preferred_element_type=jnp.float32)
        + jnp.dot(xb[...], wroot[...], preferred_element_type=jnp.float32)
        + b[...], 0.0)


def _tc_layer2_body(q0, q1, x1b, wrel, wroot, b2, w3a, w3b, b3, w4, b4,
                    w5, b5, o):
    agg = q0[...] + q1[...]
    x2 = jnp.maximum(
        jnp.dot(agg, wrel[...], preferred_element_type=jnp.float32)
        + jnp.dot(x1b[...], wroot[...], preferred_element_type=jnp.float32)
        + b2[...], 0.0)
    h = jnp.maximum(
        jnp.dot(x1b[...], w3a[...], preferred_element_type=jnp.float32)
        + jnp.dot(x2, w3b[...], preferred_element_type=jnp.float32)
        + b3[...], 0.0)
    h = jnp.maximum(
        jnp.dot(h, w4[...], preferred_element_type=jnp.float32)
        + b4[...], 0.0)
    o[...] = jnp.dot(h, w5[...], preferred_element_type=jnp.float32) + b5[...]


def _row_spec(cols):
    return pl.BlockSpec((_BLK, cols), lambda i: (i, 0))


def _full_spec(r, c):
    return pl.BlockSpec((r, c), lambda i: (0, 0))


def _tc_layer1(p0, p1, x, wrel_t, wroot_t, b):
    return pl.pallas_call(
        _tc_layer1_body,
        grid=(_N // _BLK,),
        in_specs=[
            _row_spec(_C), _row_spec(_C), _row_spec(_C),
            _full_spec(_C, _C), _full_spec(_C, _C), _full_spec(1, _C),
        ],
        out_specs=_row_spec(_C),
        out_shape=jax.ShapeDtypeStruct((_N, _C), jnp.float32),
    )(p0, p1, x, wrel_t, wroot_t, b)


def _tc_layer2(q0, q1, x1, wrel_t, wroot_t, b2, w3a, w3b, b3, w4, b4, w5, b5):
    return pl.pallas_call(
        _tc_layer2_body,
        grid=(_N // _BLK,),
        in_specs=[
            _row_spec(_C), _row_spec(_C), _row_spec(_C),
            _full_spec(_C, _C), _full_spec(_C, _C), _full_spec(1, _C),
            _full_spec(_C, 64), _full_spec(_C, 64), _full_spec(1, 64),
            _full_spec(64, 32), _full_spec(1, 32),
            _full_spec(32, 16), _full_spec(1, 16),
        ],
        out_specs=_row_spec(16),
        out_shape=jax.ShapeDtypeStruct((_N, 16), jnp.float32),
    )(q0, q1, x1, wrel_t, wroot_t, b2, w3a, w3b, b3, w4, b4, w5, b5)


def kernel(x, edge_index, W1_rel, b1_rel, W1_root, W2_rel, b2_rel, W2_root,
           W3, b3, W4, b4, W5, b5):
    src = edge_index[0]
    dst = edge_index[1]
    pad = _EPAD - _E
    pad_idx = jnp.arange(pad, dtype=jnp.int32)
    src_p = jnp.concatenate([src, pad_idx % _N])
    dst_p = jnp.concatenate([dst, _N + pad_idx % (_NPAD - _N)])
    src_r = src_p.reshape(_NW, _CHUNKS, _CHUNK)
    dst_r = dst_p.reshape(_NW, _CHUNKS, _CHUNK)
    zeros = jnp.zeros((_RPT, _C), jnp.float32)

    parts1 = _sc_agg(x, src_r, dst_r, zeros)
    x1 = _tc_layer1(parts1[0, :_N], parts1[1, :_N], x,
                    W1_rel.T, W1_root.T, b1_rel[None, :])

    parts2 = _sc_agg(x1, src_r, dst_r, zeros)
    out = _tc_layer2(parts2[0, :_N], parts2[1, :_N], x1,
                     W2_rel.T, W2_root.T, b2_rel[None, :],
                     W3[:, :_C].T, W3[:, _C:].T, b3[None, :],
                     W4.T, b4[None, :], W5.T, b5[None, :])
    return out


# speedup vs baseline: 12.0547x; 1.0440x over previous
import functools

import jax
import jax.numpy as jnp
from jax import lax
from jax.experimental import pallas as pl
from jax.experimental.pallas import tpu as pltpu
from jax.experimental.pallas import tpu_sc as plsc

_N = 10000
_C = 128
_E = 320000

_NW = 32
_CHUNK = 128
_PHASE = 40
_NPHASE = 2
_CHUNKS = _PHASE * _NPHASE
_EPW = _CHUNK * _CHUNKS
_EPAD = _NW * _EPW
_RPT = 632
_NPAD = 16 * _RPT

_BLK = 1000


def _make_sc_agg():
    mesh = plsc.VectorSubcoreMesh(core_axis_name="c", subcore_axis_name="s")

    @functools.partial(
        pl.kernel,
        mesh=mesh,
        out_type=jax.ShapeDtypeStruct((2, _NPAD, _C), jnp.float32),
        scratch_types=[
            pltpu.VMEM((_PHASE, _CHUNK), jnp.int32),
            pltpu.VMEM((_PHASE, _CHUNK), jnp.int32),
            pltpu.VMEM((_CHUNK, _C), jnp.float32),
            pltpu.VMEM((_CHUNK, _C), jnp.float32),
            pltpu.VMEM_SHARED((_NPAD, _C), jnp.float32),
            pltpu.SemaphoreType.DMA,
            pltpu.SemaphoreType.DMA,
        ],
    )
    def sc_agg(x_hbm, src_hbm, dst_hbm, zeros_hbm, out_hbm,
               sidx, didx, rows0, rows1, acc, sem0, sem1):
        cid = lax.axis_index("c")
        sid = lax.axis_index("s")
        wid = cid * 16 + sid
        pltpu.sync_copy(zeros_hbm, acc.at[pl.ds(sid * _RPT, _RPT)])
        plsc.subcore_barrier()

        for p in range(_NPHASE):
            pltpu.sync_copy(src_hbm.at[wid, pl.ds(p * _PHASE, _PHASE)], sidx)
            pltpu.sync_copy(dst_hbm.at[wid, pl.ds(p * _PHASE, _PHASE)], didx)
            pltpu.async_copy(x_hbm.at[sidx.at[0]], rows0, sem0)
            pltpu.async_copy(x_hbm.at[sidx.at[1]], rows1, sem1)

            def body(i, carry):
                j = 2 * i
                pltpu.make_async_copy(x_hbm.at[sidx.at[j]], rows0, sem0).wait()
                pltpu.sync_copy(rows0, acc.at[didx.at[j]], add=True)

                @pl.when(j + 2 < _PHASE)
                def _():
                    pltpu.async_copy(x_hbm.at[sidx.at[j + 2]], rows0, sem0)

                pltpu.make_async_copy(
                    x_hbm.at[sidx.at[j + 1]], rows1, sem1).wait()
                pltpu.sync_copy(rows1, acc.at[didx.at[j + 1]], add=True)

                @pl.when(j + 3 < _PHASE)
                def _():
                    pltpu.async_copy(x_hbm.at[sidx.at[j + 3]], rows1, sem1)
                return carry

            lax.fori_loop(0, _PHASE // 2, body, 0)
        plsc.subcore_barrier()
        pltpu.sync_copy(acc.at[pl.ds(sid * _RPT, _RPT)],
                        out_hbm.at[cid, pl.ds(sid * _RPT, _RPT)])

    return sc_agg


_sc_agg = _make_sc_agg()


def _tc_layer1_body(p0, p1, xb, wrel, wroot, b, o):
    agg = p0[0] + p1[0]
    o[...] = jnp.maximum(
        jnp.dot(agg, wrel[...], preferred_element_type=jnp.float32)
        + jnp.dot(xb[...], wroot[...], preferred_element_type=jnp.float32)
        + b[...], 0.0)


def _tc_layer2_body(q0, q1, x1b, wrel, wroot, b2, w3a, w3b, b3, w4, b4,
                    w5, b5, o):
    agg = q0[0] + q1[0]
    x2 = jnp.maximum(
        jnp.dot(agg, wrel[...], preferred_element_type=jnp.float32)
        + jnp.dot(x1b[...], wroot[...], preferred_element_type=jnp.float32)
        + b2[...], 0.0)
    h = jnp.maximum(
        jnp.dot(x1b[...], w3a[...], preferred_element_type=jnp.float32)
        + jnp.dot(x2, w3b[...], preferred_element_type=jnp.float32)
        + b3[...], 0.0)
    h = jnp.maximum(
        jnp.dot(h, w4[...], preferred_element_type=jnp.float32)
        + b4[...], 0.0)
    o[...] = jnp.dot(h, w5[...], preferred_element_type=jnp.float32) + b5[...]


def _row_spec(cols):
    return pl.BlockSpec((_BLK, cols), lambda i: (i, 0))


def _part_spec(k):
    return pl.BlockSpec((1, _BLK, _C), lambda i, _k=k: (_k, i, 0))


def _full_spec(r, c):
    return pl.BlockSpec((r, c), lambda i: (0, 0))


def _tc_layer1(parts, x, wrel_t, wroot_t, b):
    return pl.pallas_call(
        _tc_layer1_body,
        grid=(_N // _BLK,),
        in_specs=[
            _part_spec(0), _part_spec(1), _row_spec(_C),
            _full_spec(_C, _C), _full_spec(_C, _C), _full_spec(1, _C),
        ],
        out_specs=_row_spec(_C),
        out_shape=jax.ShapeDtypeStruct((_N, _C), jnp.float32),
    )(parts, parts, x, wrel_t, wroot_t, b)


def _tc_layer2(parts, x1, wrel_t, wroot_t, b2, w3a, w3b, b3, w4, b4, w5, b5):
    return pl.pallas_call(
        _tc_layer2_body,
        grid=(_N // _BLK,),
        in_specs=[
            _part_spec(0), _part_spec(1), _row_spec(_C),
            _full_spec(_C, _C), _full_spec(_C, _C), _full_spec(1, _C),
            _full_spec(_C, 64), _full_spec(_C, 64), _full_spec(1, 64),
            _full_spec(64, 32), _full_spec(1, 32),
            _full_spec(32, 16), _full_spec(1, 16),
        ],
        out_specs=_row_spec(16),
        out_shape=jax.ShapeDtypeStruct((_N, 16), jnp.float32),
    )(parts, parts, x1, wrel_t, wroot_t, b2, w3a, w3b, b3, w4, b4, w5, b5)


def kernel(x, edge_index, W1_rel, b1_rel, W1_root, W2_rel, b2_rel, W2_root,
           W3, b3, W4, b4, W5, b5):
    src = edge_index[0]
    dst = edge_index[1]
    pad = _EPAD - _E
    pad_idx = jnp.arange(pad, dtype=jnp.int32)
    src_p = jnp.concatenate([src, pad_idx % _N])
    dst_p = jnp.concatenate([dst, _N + pad_idx % (_NPAD - _N)])
    src_r = src_p.reshape(_NW, _CHUNKS, _CHUNK)
    dst_r = dst_p.reshape(_NW, _CHUNKS, _CHUNK)
    zeros = jnp.zeros((_RPT, _C), jnp.float32)

    parts1 = _sc_agg(x, src_r, dst_r, zeros)
    x1 = _tc_layer1(parts1, x, W1_rel.T, W1_root.T, b1_rel[None, :])

    parts2 = _sc_agg(x1, src_r, dst_r, zeros)
    out = _tc_layer2(parts2, x1,
                     W2_rel.T, W2_root.T, b2_rel[None, :],
                     W3[:, :_C].T, W3[:, _C:].T, b3[None, :],
                     W4.T, b4[None, :], W5.T, b5[None, :])
    return out
